# baseline (speedup 1.0000x reference)
import functools

import jax
import jax.numpy as jnp
from jax import lax
from jax.experimental import pallas as pl
from jax.experimental.pallas import tpu as pltpu

_EPS = 1e-5


def _lex_less(ka, pa, kb, pb):
    return (ka < kb) | ((ka == kb) & (pa < pb))


def _flip_blocks(v, r, jj):
    d = v.shape[-1]
    v4 = v.reshape(r // (2 * jj), 2, jj, d)
    return jnp.concatenate([v4[:, 1], v4[:, 0]], axis=1).reshape(r, d)


def _reg_finish(kv, pv, r, j0, up):
    iota = lax.broadcasted_iota(jnp.int32, (r, 1), 0)
    jj = j0
    while jj >= 1:
        mh = (iota & jj) != 0
        pk = _flip_blocks(kv, r, jj)
        pp = _flip_blocks(pv, r, jj)
        lp = _lex_less(pk, pp, kv, pv)
        tp = jnp.logical_xor(jnp.logical_xor(lp, mh), jnp.logical_not(up))
        kv = jnp.where(tp, pk, kv)
        pv = jnp.where(tp, pp, pv)
        jj //= 2
    return kv, pv


def _init_pass(keys_ref, pay_ref, h, r, mean, den, w, b):
    logr = r.bit_length() - 1
    iota = lax.broadcasted_iota(jnp.int32, (r, 1), 0)

    def body(t, carry):
        r0 = t * r
        rows = pl.ds(r0, r)
        kv = ((keys_ref[rows, :] - mean) / den) * w + b
        pv = pay_ref[rows, :]
        for s in range(1, logr + 1):
            k = 1 << s
            up = (((iota + r0) & k) == 0)
            kv, pv = _reg_finish(kv, pv, r, k // 2, up)
        keys_ref[rows, :] = kv
        pay_ref[rows, :] = pv
        return carry

    lax.fori_loop(0, h // r, body, 0)


def _far_pass(keys_ref, pay_ref, h, r, j, k, flip, fuse_near):
    nbt = j // r
    npair = (h // (2 * j)) * nbt
    unroll = 2 if (npair % 2 == 0 and not fuse_near) else 1

    def one_pair(t):
        off = t & (nbt - 1)
        blk = t >> (nbt.bit_length() - 1)
        row_a = blk * (2 * j) + off * r
        row_b = row_a + j
        up = ((row_a & k) == 0) != flip
        sa = pl.ds(row_a, r)
        sb = pl.ds(row_b, r)
        ka = keys_ref[sa, :]
        kb = keys_ref[sb, :]
        pa = pay_ref[sa, :]
        pb = pay_ref[sb, :]
        less = _lex_less(kb, pb, ka, pa)
        swap = jnp.logical_xor(less, jnp.logical_not(up))
        nka = jnp.where(swap, kb, ka)
        nkb = jnp.where(swap, ka, kb)
        npa = jnp.where(swap, pb, pa)
        npb = jnp.where(swap, pa, pb)
        if fuse_near:
            upm = jnp.broadcast_to(up, (r, 1))
            nka, npa = _reg_finish(nka, npa, r, r // 2, upm)
            nkb, npb = _reg_finish(nkb, npb, r, r // 2, upm)
        keys_ref[sa, :] = nka
        keys_ref[sb, :] = nkb
        pay_ref[sa, :] = npa
        pay_ref[sb, :] = npb

    def body(t, carry):
        for u in range(unroll):
            one_pair(t * unroll + u)
        return carry

    lax.fori_loop(0, npair // unroll, body, 0)


def _merge_region(keys_ref, pay_ref, h, r, k, flip):
    j = h >> 1
    while j > r:
        _far_pass(keys_ref, pay_ref, h, r, j, k, flip, False)
        j >>= 1
    if j == r:
        _far_pass(keys_ref, pay_ref, h, r, r, k, flip, True)
    else:
        iota = lax.broadcasted_iota(jnp.int32, (h, 1), 0)

        def body(t, carry):
            kv = keys_ref[...]
            pv = pay_ref[...]
            up = ((iota & k) == 0) != flip
            kv, pv = _reg_finish(kv, pv, h, j, up)
            keys_ref[...] = kv
            pay_ref[...] = pv
            return carry

        lax.fori_loop(0, 1, body, 0)


def _phase_a_impl(h, r, x_ref, m_ref, v_ref, w_ref, b_ref, gb_ref,
                  keys_out, pay_out, kv_ref, pv_ref, sem1, sem2):
    hh = pl.program_id(0)
    base = hh * h
    c1 = pltpu.make_async_copy(x_ref.at[pl.ds(base, h)], kv_ref, sem1)
    c2 = pltpu.make_async_copy(gb_ref.at[pl.ds(base, h)], pv_ref, sem2)
    c1.start()
    c2.start()
    c1.wait()
    c2.wait()

    den = jnp.sqrt(v_ref[...] + _EPS)
    _init_pass(kv_ref, pv_ref, h, r, m_ref[...], den, w_ref[...], b_ref[...])

    logr = r.bit_length() - 1
    logh = h.bit_length() - 1
    for s in range(logr + 1, logh + 1):
        k = 1 << s
        flip = (hh == 1) if k == h else False
        j = k >> 1
        while j > r:
            _far_pass(kv_ref, pv_ref, h, r, j, k, flip, False)
            j >>= 1
        _far_pass(kv_ref, pv_ref, h, r, r, k, flip, True)

    o1 = pltpu.make_async_copy(kv_ref, keys_out.at[pl.ds(base, h)], sem1)
    o2 = pltpu.make_async_copy(pv_ref, pay_out.at[pl.ds(base, h)], sem2)
    o1.start()
    o2.start()
    o1.wait()
    o2.wait()


def _phase_c_impl(h, r, cb, keys_ref, pay_ref, out_ref,
                  kv_ref, pv_ref, pk_ref, pp_ref, sem1, sem2, psems):
    hh = pl.program_id(0)
    base = hh * h
    pbase = (1 - hh) * h
    c1 = pltpu.make_async_copy(keys_ref.at[pl.ds(base, h)], kv_ref, sem1)
    c2 = pltpu.make_async_copy(pay_ref.at[pl.ds(base, h)], pv_ref, sem2)
    c1.start()
    c2.start()
    c1.wait()
    c2.wait()

    nch = h // cb
    is_hi = hh == 1

    def fetch_copies(c, buf):
        rows = pl.ds(pbase + c * cb, cb)
        k_c = pltpu.make_async_copy(keys_ref.at[rows], pk_ref.at[buf],
                                    psems.at[buf, 0])
        p_c = pltpu.make_async_copy(pay_ref.at[rows], pp_ref.at[buf],
                                    psems.at[buf, 1])
        return k_c, p_c

    def start_fetch(c, buf):
        k_c, p_c = fetch_copies(c, buf)
        k_c.start()
        p_c.start()

    def wait_fetch(c, buf):
        k_c, p_c = fetch_copies(c, buf)
        k_c.wait()
        p_c.wait()

    start_fetch(0, 0)

    def cross_body(c, carry):
        buf = lax.rem(c, 2)
        nbuf = lax.rem(c + 1, 2)

        @pl.when(c + 1 < nch)
        def _():
            start_fetch(c + 1, nbuf)

        wait_fetch(c, buf)
        nt = cb // r

        def tile_body(t, carry2):
            rows = pl.ds(c * cb + t * r, r)
            prow = pl.ds(t * r, r)
            ko = kv_ref[rows, :]
            po = pv_ref[rows, :]
            kp = pk_ref[buf, prow, :]
            pp = pp_ref[buf, prow, :]
            take = jnp.logical_xor(_lex_less(kp, pp, ko, po), is_hi)
            kv_ref[rows, :] = jnp.where(take, kp, ko)
            pv_ref[rows, :] = jnp.where(take, pp, po)
            return carry2

        lax.fori_loop(0, nt, tile_body, 0)
        return carry

    lax.fori_loop(0, nch, cross_body, 0)

    _merge_region(kv_ref, pv_ref, h, r, 2 * h, False)

    o2 = pltpu.make_async_copy(pv_ref, out_ref.at[pl.ds(base, h)], sem2)
    o2.start()
    o2.wait()


def _run2(x, bn_weight, bn_bias, running_mean, running_var, gauss_point,
          r=64, cb=2048, interpret=False):
    n, d = x.shape
    h = n // 2
    f32 = jnp.float32
    gb = jnp.broadcast_to(gauss_point[:, None], (n, d))
    m2 = running_mean.reshape(1, d)
    v2 = running_var.reshape(1, d)
    w2 = bn_weight.reshape(1, d)
    b2 = bn_bias.reshape(1, d)

    hbm = pl.BlockSpec(memory_space=pltpu.MemorySpace.HBM)
    vsmall = pl.BlockSpec((1, d), lambda hh: (0, 0))

    keys1, pay1 = pl.pallas_call(
        functools.partial(_phase_a_impl, h, r),
        grid=(2,),
        in_specs=[hbm, vsmall, vsmall, vsmall, vsmall, hbm],
        out_specs=[hbm, hbm],
        out_shape=[jax.ShapeDtypeStruct((n, d), f32),
                   jax.ShapeDtypeStruct((n, d), f32)],
        scratch_shapes=[pltpu.VMEM((h, d), f32), pltpu.VMEM((h, d), f32),
                        pltpu.SemaphoreType.DMA, pltpu.SemaphoreType.DMA],
        compiler_params=pltpu.CompilerParams(
            dimension_semantics=("arbitrary",),
        ),
        interpret=interpret,
    )(x, m2, v2, w2, b2, gb)

    out = pl.pallas_call(
        functools.partial(_phase_c_impl, h, r, cb),
        grid=(2,),
        in_specs=[hbm, hbm],
        out_specs=hbm,
        out_shape=jax.ShapeDtypeStruct((n, d), f32),
        scratch_shapes=[pltpu.VMEM((h, d), f32), pltpu.VMEM((h, d), f32),
                        pltpu.VMEM((2, cb, d), f32),
                        pltpu.VMEM((2, cb, d), f32),
                        pltpu.SemaphoreType.DMA, pltpu.SemaphoreType.DMA,
                        pltpu.SemaphoreType.DMA((2, 2))],
        compiler_params=pltpu.CompilerParams(
            dimension_semantics=("arbitrary",),
        ),
        interpret=interpret,
    )(keys1, pay1)
    return out


def kernel(x, bn_weight, bn_bias, running_mean, running_var, gauss_point):
    return _run2(x, bn_weight, bn_bias, running_mean, running_var,
                 gauss_point)


# speedup vs baseline: 1.7296x; 1.7296x over previous
import functools

import jax
import jax.numpy as jnp
from jax import lax
from jax.experimental import pallas as pl
from jax.experimental.pallas import tpu as pltpu

_EPS = 1e-5


def _lex_less(ka, pa, kb, pb):
    return (ka < kb) | ((ka == kb) & (pa < pb))


def _reg_finish(kv, pv, r, j0, up):
    iota = lax.broadcasted_iota(jnp.int32, (r, 1), 0)
    jj = j0
    while jj >= 1:
        mh = (iota & jj) != 0
        kd = jnp.concatenate([kv[jj:], kv[:jj]], axis=0)
        ku = jnp.concatenate([kv[r - jj:], kv[:r - jj]], axis=0)
        pd = jnp.concatenate([pv[jj:], pv[:jj]], axis=0)
        pu = jnp.concatenate([pv[r - jj:], pv[:r - jj]], axis=0)
        pk = jnp.where(mh, ku, kd)
        pp = jnp.where(mh, pu, pd)
        lp = _lex_less(pk, pp, kv, pv)
        tp = jnp.logical_xor(jnp.logical_xor(lp, mh), jnp.logical_not(up))
        kv = jnp.where(tp, pk, kv)
        pv = jnp.where(tp, pp, pv)
        jj //= 2
    return kv, pv


def _init_pass(keys_ref, pay_ref, h, r, mean, den, w, b):
    logr = r.bit_length() - 1
    iota = lax.broadcasted_iota(jnp.int32, (r, 1), 0)

    def body(t, carry):
        r0 = t * r
        rows = pl.ds(r0, r)
        kv = ((keys_ref[rows, :] - mean) / den) * w + b
        pv = pay_ref[rows, :]
        for s in range(1, logr + 1):
            k = 1 << s
            up = (((iota + r0) & k) == 0)
            kv, pv = _reg_finish(kv, pv, r, k // 2, up)
        keys_ref[rows, :] = kv
        pay_ref[rows, :] = pv
        return carry

    lax.fori_loop(0, h // r, body, 0)


def _far_pass(keys_ref, pay_ref, h, r, j, k, flip, fuse_near):
    nbt = j // r
    npair = (h // (2 * j)) * nbt
    unroll = 2 if (npair % 2 == 0 and not fuse_near) else 1

    def one_pair(t):
        off = t & (nbt - 1)
        blk = t >> (nbt.bit_length() - 1)
        row_a = blk * (2 * j) + off * r
        row_b = row_a + j
        up = ((row_a & k) == 0) != flip
        sa = pl.ds(row_a, r)
        sb = pl.ds(row_b, r)
        ka = keys_ref[sa, :]
        kb = keys_ref[sb, :]
        pa = pay_ref[sa, :]
        pb = pay_ref[sb, :]
        less = _lex_less(kb, pb, ka, pa)
        swap = jnp.logical_xor(less, jnp.logical_not(up))
        nka = jnp.where(swap, kb, ka)
        nkb = jnp.where(swap, ka, kb)
        npa = jnp.where(swap, pb, pa)
        npb = jnp.where(swap, pa, pb)
        if fuse_near:
            upm = jnp.broadcast_to(up, (r, 1))
            nka, npa = _reg_finish(nka, npa, r, r // 2, upm)
            nkb, npb = _reg_finish(nkb, npb, r, r // 2, upm)
        keys_ref[sa, :] = nka
        keys_ref[sb, :] = nkb
        pay_ref[sa, :] = npa
        pay_ref[sb, :] = npb

    def body(t, carry):
        for u in range(unroll):
            one_pair(t * unroll + u)
        return carry

    lax.fori_loop(0, npair // unroll, body, 0)


def _merge_region(keys_ref, pay_ref, h, r, k, flip):
    j = h >> 1
    while j > r:
        _far_pass(keys_ref, pay_ref, h, r, j, k, flip, False)
        j >>= 1
    if j == r:
        _far_pass(keys_ref, pay_ref, h, r, r, k, flip, True)
    else:
        iota = lax.broadcasted_iota(jnp.int32, (h, 1), 0)

        def body(t, carry):
            kv = keys_ref[...]
            pv = pay_ref[...]
            up = ((iota & k) == 0) != flip
            kv, pv = _reg_finish(kv, pv, h, j, up)
            keys_ref[...] = kv
            pay_ref[...] = pv
            return carry

        lax.fori_loop(0, 1, body, 0)


def _phase_a_impl(h, r, x_ref, m_ref, v_ref, w_ref, b_ref, gb_ref,
                  keys_out, pay_out, kv_ref, pv_ref, sem1, sem2):
    hh = pl.program_id(0)
    base = hh * h
    c1 = pltpu.make_async_copy(x_ref.at[pl.ds(base, h)], kv_ref, sem1)
    c2 = pltpu.make_async_copy(gb_ref.at[pl.ds(base, h)], pv_ref, sem2)
    c1.start()
    c2.start()
    c1.wait()
    c2.wait()

    den = jnp.sqrt(v_ref[...] + _EPS)
    _init_pass(kv_ref, pv_ref, h, r, m_ref[...], den, w_ref[...], b_ref[...])

    logr = r.bit_length() - 1
    logh = h.bit_length() - 1
    for s in range(logr + 1, logh + 1):
        k = 1 << s
        flip = (hh == 1) if k == h else False
        j = k >> 1
        while j > r:
            _far_pass(kv_ref, pv_ref, h, r, j, k, flip, False)
            j >>= 1
        _far_pass(kv_ref, pv_ref, h, r, r, k, flip, True)

    o1 = pltpu.make_async_copy(kv_ref, keys_out.at[pl.ds(base, h)], sem1)
    o2 = pltpu.make_async_copy(pv_ref, pay_out.at[pl.ds(base, h)], sem2)
    o1.start()
    o2.start()
    o1.wait()
    o2.wait()


def _phase_c_impl(h, r, cb, keys_ref, pay_ref, out_ref,
                  kv_ref, pv_ref, pk_ref, pp_ref, sem1, sem2, psems):
    hh = pl.program_id(0)
    base = hh * h
    pbase = (1 - hh) * h
    c1 = pltpu.make_async_copy(keys_ref.at[pl.ds(base, h)], kv_ref, sem1)
    c2 = pltpu.make_async_copy(pay_ref.at[pl.ds(base, h)], pv_ref, sem2)
    c1.start()
    c2.start()
    c1.wait()
    c2.wait()

    nch = h // cb
    is_hi = hh == 1

    def fetch_copies(c, buf):
        rows = pl.ds(pbase + c * cb, cb)
        k_c = pltpu.make_async_copy(keys_ref.at[rows], pk_ref.at[buf],
                                    psems.at[buf, 0])
        p_c = pltpu.make_async_copy(pay_ref.at[rows], pp_ref.at[buf],
                                    psems.at[buf, 1])
        return k_c, p_c

    def start_fetch(c, buf):
        k_c, p_c = fetch_copies(c, buf)
        k_c.start()
        p_c.start()

    def wait_fetch(c, buf):
        k_c, p_c = fetch_copies(c, buf)
        k_c.wait()
        p_c.wait()

    start_fetch(0, 0)

    def cross_body(c, carry):
        buf = lax.rem(c, 2)
        nbuf = lax.rem(c + 1, 2)

        @pl.when(c + 1 < nch)
        def _():
            start_fetch(c + 1, nbuf)

        wait_fetch(c, buf)
        nt = cb // r

        def tile_body(t, carry2):
            rows = pl.ds(c * cb + t * r, r)
            prow = pl.ds(t * r, r)
            ko = kv_ref[rows, :]
            po = pv_ref[rows, :]
            kp = pk_ref[buf, prow, :]
            pp = pp_ref[buf, prow, :]
            take = jnp.logical_xor(_lex_less(kp, pp, ko, po), is_hi)
            kv_ref[rows, :] = jnp.where(take, kp, ko)
            pv_ref[rows, :] = jnp.where(take, pp, po)
            return carry2

        lax.fori_loop(0, nt, tile_body, 0)
        return carry

    lax.fori_loop(0, nch, cross_body, 0)

    _merge_region(kv_ref, pv_ref, h, r, 2 * h, False)

    o2 = pltpu.make_async_copy(pv_ref, out_ref.at[pl.ds(base, h)], sem2)
    o2.start()
    o2.wait()


def _run2(x, bn_weight, bn_bias, running_mean, running_var, gauss_point,
          r=64, cb=2048, interpret=False):
    n, d = x.shape
    h = n // 2
    f32 = jnp.float32
    gb = jnp.broadcast_to(gauss_point[:, None], (n, d))
    m2 = running_mean.reshape(1, d)
    v2 = running_var.reshape(1, d)
    w2 = bn_weight.reshape(1, d)
    b2 = bn_bias.reshape(1, d)

    hbm = pl.BlockSpec(memory_space=pltpu.MemorySpace.HBM)
    vsmall = pl.BlockSpec((1, d), lambda hh: (0, 0))

    keys1, pay1 = pl.pallas_call(
        functools.partial(_phase_a_impl, h, r),
        grid=(2,),
        in_specs=[hbm, vsmall, vsmall, vsmall, vsmall, hbm],
        out_specs=[hbm, hbm],
        out_shape=[jax.ShapeDtypeStruct((n, d), f32),
                   jax.ShapeDtypeStruct((n, d), f32)],
        scratch_shapes=[pltpu.VMEM((h, d), f32), pltpu.VMEM((h, d), f32),
                        pltpu.SemaphoreType.DMA, pltpu.SemaphoreType.DMA],
        compiler_params=pltpu.CompilerParams(
            dimension_semantics=("arbitrary",),
        ),
        interpret=interpret,
    )(x, m2, v2, w2, b2, gb)

    out = pl.pallas_call(
        functools.partial(_phase_c_impl, h, r, cb),
        grid=(2,),
        in_specs=[hbm, hbm],
        out_specs=hbm,
        out_shape=jax.ShapeDtypeStruct((n, d), f32),
        scratch_shapes=[pltpu.VMEM((h, d), f32), pltpu.VMEM((h, d), f32),
                        pltpu.VMEM((2, cb, d), f32),
                        pltpu.VMEM((2, cb, d), f32),
                        pltpu.SemaphoreType.DMA, pltpu.SemaphoreType.DMA,
                        pltpu.SemaphoreType.DMA((2, 2))],
        compiler_params=pltpu.CompilerParams(
            dimension_semantics=("arbitrary",),
        ),
        interpret=interpret,
    )(keys1, pay1)
    return out


def kernel(x, bn_weight, bn_bias, running_mean, running_var, gauss_point):
    return _run2(x, bn_weight, bn_bias, running_mean, running_var,
                 gauss_point)


# speedup vs baseline: 2.0496x; 1.1850x over previous
import functools

import jax
import jax.numpy as jnp
from jax import lax
from jax.experimental import pallas as pl
from jax.experimental.pallas import tpu as pltpu

_EPS = 1e-5


def _lex_less(ka, pa, kb, pb):
    return (ka < kb) | ((ka == kb) & (pa < pb))


def _partner(v, r, jj, mh):
    d = v.shape[-1]
    if jj < 8:
        v3 = v.reshape(r // 8, 8, d)
        dn = jnp.concatenate([v3[:, jj:], v3[:, :jj]], axis=1)
        if jj == 4:
            return dn.reshape(r, d)
        uprot = jnp.concatenate([v3[:, 8 - jj:], v3[:, :8 - jj]], axis=1)
        return jnp.where(mh.reshape(r // 8, 8, 1), uprot, dn).reshape(r, d)
    vd = jnp.concatenate([v[jj:], v[:jj]], axis=0)
    vu = jnp.concatenate([v[r - jj:], v[:r - jj]], axis=0)
    return jnp.where(mh, vu, vd)


def _reg_finish(kv, pv, r, j0, up):
    iota = lax.broadcasted_iota(jnp.int32, (r, 1), 0)
    jj = j0
    while jj >= 1:
        mh = (iota & jj) != 0
        pk = _partner(kv, r, jj, mh)
        pp = _partner(pv, r, jj, mh)
        lp = _lex_less(pk, pp, kv, pv)
        tp = jnp.logical_xor(jnp.logical_xor(lp, mh), jnp.logical_not(up))
        kv = jnp.where(tp, pk, kv)
        pv = jnp.where(tp, pp, pv)
        jj //= 2
    return kv, pv


def _init_pass(keys_ref, pay_ref, h, r, mean, den, w, b):
    logr = r.bit_length() - 1
    iota = lax.broadcasted_iota(jnp.int32, (r, 1), 0)

    def body(t, carry):
        r0 = t * r
        rows = pl.ds(r0, r)
        kv = ((keys_ref[rows, :] - mean) / den) * w + b
        pv = pay_ref[rows, :]
        for s in range(1, logr + 1):
            k = 1 << s
            up = (((iota + r0) & k) == 0)
            kv, pv = _reg_finish(kv, pv, r, k // 2, up)
        keys_ref[rows, :] = kv
        pay_ref[rows, :] = pv
        return carry

    lax.fori_loop(0, h // r, body, 0)


def _far_pass(keys_ref, pay_ref, h, r, j, k, flip, fuse_near):
    nbt = j // r
    npair = (h // (2 * j)) * nbt

    def one_pair(t):
        off = t & (nbt - 1)
        blk = t >> (nbt.bit_length() - 1)
        row_a = blk * (2 * j) + off * r
        row_b = row_a + j
        up = ((row_a & k) == 0) != flip
        sa = pl.ds(row_a, r)
        sb = pl.ds(row_b, r)
        ka = keys_ref[sa, :]
        kb = keys_ref[sb, :]
        pa = pay_ref[sa, :]
        pb = pay_ref[sb, :]
        less = _lex_less(kb, pb, ka, pa)
        swap = jnp.logical_xor(less, jnp.logical_not(up))
        nka = jnp.where(swap, kb, ka)
        nkb = jnp.where(swap, ka, kb)
        npa = jnp.where(swap, pb, pa)
        npb = jnp.where(swap, pa, pb)
        if fuse_near:
            upm = jnp.broadcast_to(up, (r, 1))
            nka, npa = _reg_finish(nka, npa, r, r // 2, upm)
            nkb, npb = _reg_finish(nkb, npb, r, r // 2, upm)
        keys_ref[sa, :] = nka
        keys_ref[sb, :] = nkb
        pay_ref[sa, :] = npa
        pay_ref[sb, :] = npb

    def body(t, carry):
        one_pair(t)
        return carry

    lax.fori_loop(0, npair, body, 0)


def _merge_region(keys_ref, pay_ref, h, r, k, flip):
    j = h >> 1
    while j > r:
        _far_pass(keys_ref, pay_ref, h, r, j, k, flip, False)
        j >>= 1
    if j == r:
        _far_pass(keys_ref, pay_ref, h, r, r, k, flip, True)
    else:
        iota = lax.broadcasted_iota(jnp.int32, (h, 1), 0)

        def body(t, carry):
            kv = keys_ref[...]
            pv = pay_ref[...]
            up = ((iota & k) == 0) != flip
            kv, pv = _reg_finish(kv, pv, h, j, up)
            keys_ref[...] = kv
            pay_ref[...] = pv
            return carry

        lax.fori_loop(0, 1, body, 0)


def _phase_a_impl(h, r, x_ref, m_ref, v_ref, w_ref, b_ref, gb_ref,
                  keys_out, pay_out, kv_ref, pv_ref, sem1, sem2):
    hh = pl.program_id(0)
    base = hh * h
    c1 = pltpu.make_async_copy(x_ref.at[pl.ds(base, h)], kv_ref, sem1)
    c2 = pltpu.make_async_copy(gb_ref.at[pl.ds(base, h)], pv_ref, sem2)
    c1.start()
    c2.start()
    c1.wait()
    c2.wait()

    den = jnp.sqrt(v_ref[...] + _EPS)
    _init_pass(kv_ref, pv_ref, h, r, m_ref[...], den, w_ref[...], b_ref[...])

    logr = r.bit_length() - 1
    logh = h.bit_length() - 1
    for s in range(logr + 1, logh + 1):
        k = 1 << s
        flip = (hh == 1) if k == h else False
        j = k >> 1
        while j > r:
            _far_pass(kv_ref, pv_ref, h, r, j, k, flip, False)
            j >>= 1
        _far_pass(kv_ref, pv_ref, h, r, r, k, flip, True)

    o1 = pltpu.make_async_copy(kv_ref, keys_out.at[pl.ds(base, h)], sem1)
    o2 = pltpu.make_async_copy(pv_ref, pay_out.at[pl.ds(base, h)], sem2)
    o1.start()
    o2.start()
    o1.wait()
    o2.wait()


def _phase_c_impl(h, r, cb, keys_ref, pay_ref, out_ref,
                  kv_ref, pv_ref, pk_ref, pp_ref, sem1, sem2, psems):
    hh = pl.program_id(0)
    base = hh * h
    pbase = (1 - hh) * h
    c1 = pltpu.make_async_copy(keys_ref.at[pl.ds(base, h)], kv_ref, sem1)
    c2 = pltpu.make_async_copy(pay_ref.at[pl.ds(base, h)], pv_ref, sem2)
    c1.start()
    c2.start()
    c1.wait()
    c2.wait()

    nch = h // cb
    is_hi = hh == 1

    def fetch_copies(c, buf):
        rows = pl.ds(pbase + c * cb, cb)
        k_c = pltpu.make_async_copy(keys_ref.at[rows], pk_ref.at[buf],
                                    psems.at[buf, 0])
        p_c = pltpu.make_async_copy(pay_ref.at[rows], pp_ref.at[buf],
                                    psems.at[buf, 1])
        return k_c, p_c

    def start_fetch(c, buf):
        k_c, p_c = fetch_copies(c, buf)
        k_c.start()
        p_c.start()

    def wait_fetch(c, buf):
        k_c, p_c = fetch_copies(c, buf)
        k_c.wait()
        p_c.wait()

    start_fetch(0, 0)

    def cross_body(c, carry):
        buf = lax.rem(c, 2)
        nbuf = lax.rem(c + 1, 2)

        @pl.when(c + 1 < nch)
        def _():
            start_fetch(c + 1, nbuf)

        wait_fetch(c, buf)
        nt = cb // r

        def tile_body(t, carry2):
            rows = pl.ds(c * cb + t * r, r)
            prow = pl.ds(t * r, r)
            ko = kv_ref[rows, :]
            po = pv_ref[rows, :]
            kp = pk_ref[buf, prow, :]
            pp = pp_ref[buf, prow, :]
            take = jnp.logical_xor(_lex_less(kp, pp, ko, po), is_hi)
            kv_ref[rows, :] = jnp.where(take, kp, ko)
            pv_ref[rows, :] = jnp.where(take, pp, po)
            return carry2

        lax.fori_loop(0, nt, tile_body, 0)
        return carry

    lax.fori_loop(0, nch, cross_body, 0)

    _merge_region(kv_ref, pv_ref, h, r, 2 * h, False)

    o2 = pltpu.make_async_copy(pv_ref, out_ref.at[pl.ds(base, h)], sem2)
    o2.start()
    o2.wait()


def _run2(x, bn_weight, bn_bias, running_mean, running_var, gauss_point,
          r=64, cb=2048, interpret=False):
    n, d = x.shape
    h = n // 2
    f32 = jnp.float32
    gb = jnp.broadcast_to(gauss_point[:, None], (n, d))
    m2 = running_mean.reshape(1, d)
    v2 = running_var.reshape(1, d)
    w2 = bn_weight.reshape(1, d)
    b2 = bn_bias.reshape(1, d)

    hbm = pl.BlockSpec(memory_space=pltpu.MemorySpace.HBM)
    vsmall = pl.BlockSpec((1, d), lambda hh: (0, 0))

    keys1, pay1 = pl.pallas_call(
        functools.partial(_phase_a_impl, h, r),
        grid=(2,),
        in_specs=[hbm, vsmall, vsmall, vsmall, vsmall, hbm],
        out_specs=[hbm, hbm],
        out_shape=[jax.ShapeDtypeStruct((n, d), f32),
                   jax.ShapeDtypeStruct((n, d), f32)],
        scratch_shapes=[pltpu.VMEM((h, d), f32), pltpu.VMEM((h, d), f32),
                        pltpu.SemaphoreType.DMA, pltpu.SemaphoreType.DMA],
        compiler_params=pltpu.CompilerParams(
            dimension_semantics=("arbitrary",),
        ),
        interpret=interpret,
    )(x, m2, v2, w2, b2, gb)

    out = pl.pallas_call(
        functools.partial(_phase_c_impl, h, r, cb),
        grid=(2,),
        in_specs=[hbm, hbm],
        out_specs=hbm,
        out_shape=jax.ShapeDtypeStruct((n, d), f32),
        scratch_shapes=[pltpu.VMEM((h, d), f32), pltpu.VMEM((h, d), f32),
                        pltpu.VMEM((2, cb, d), f32),
                        pltpu.VMEM((2, cb, d), f32),
                        pltpu.SemaphoreType.DMA, pltpu.SemaphoreType.DMA,
                        pltpu.SemaphoreType.DMA((2, 2))],
        compiler_params=pltpu.CompilerParams(
            dimension_semantics=("arbitrary",),
        ),
        interpret=interpret,
    )(keys1, pay1)
    return out


def kernel(x, bn_weight, bn_bias, running_mean, running_var, gauss_point):
    return _run2(x, bn_weight, bn_bias, running_mean, running_var,
                 gauss_point)


# speedup vs baseline: 2.0851x; 1.0173x over previous
import functools

import jax
import jax.numpy as jnp
from jax import lax
from jax.experimental import pallas as pl
from jax.experimental.pallas import tpu as pltpu

_EPS = 1e-5


def _lex_less(ka, pa, kb, pb):
    return (ka < kb) | ((ka == kb) & (pa < pb))


def _partner(v, r, jj, mh):
    d = v.shape[-1]
    if jj < 8:
        v3 = v.reshape(r // 8, 8, d)
        dn = jnp.concatenate([v3[:, jj:], v3[:, :jj]], axis=1)
        if jj == 4:
            return dn.reshape(r, d)
        uprot = jnp.concatenate([v3[:, 8 - jj:], v3[:, :8 - jj]], axis=1)
        return jnp.where(mh.reshape(r // 8, 8, 1), uprot, dn).reshape(r, d)
    vd = jnp.concatenate([v[jj:], v[:jj]], axis=0)
    vu = jnp.concatenate([v[r - jj:], v[:r - jj]], axis=0)
    return jnp.where(mh, vu, vd)


def _reg_finish(kv, pv, r, j0, up):
    iota = lax.broadcasted_iota(jnp.int32, (r, 1), 0)
    jj = j0
    while jj >= 1:
        mh = (iota & jj) != 0
        pk = _partner(kv, r, jj, mh)
        pp = _partner(pv, r, jj, mh)
        lp = _lex_less(pk, pp, kv, pv)
        tp = jnp.logical_xor(jnp.logical_xor(lp, mh), jnp.logical_not(up))
        kv = jnp.where(tp, pk, kv)
        pv = jnp.where(tp, pp, pv)
        jj //= 2
    return kv, pv


def _init_pass(keys_ref, pay_ref, h, r, mean, den, w, b):
    logr = r.bit_length() - 1
    iota = lax.broadcasted_iota(jnp.int32, (r, 1), 0)

    def body(t, carry):
        r0 = t * r
        rows = pl.ds(r0, r)
        kv = ((keys_ref[rows, :] - mean) / den) * w + b
        pv = pay_ref[rows, :]
        for s in range(1, logr + 1):
            k = 1 << s
            up = (((iota + r0) & k) == 0)
            kv, pv = _reg_finish(kv, pv, r, k // 2, up)
        keys_ref[rows, :] = kv
        pay_ref[rows, :] = pv
        return carry

    lax.fori_loop(0, h // r, body, 0)


def _far_pass(keys_ref, pay_ref, h, r, j, k, flip, fuse_near):
    nbt = j // r
    npair = (h // (2 * j)) * nbt

    def one_pair(t):
        off = t & (nbt - 1)
        blk = t >> (nbt.bit_length() - 1)
        row_a = blk * (2 * j) + off * r
        row_b = row_a + j
        up = ((row_a & k) == 0) != flip
        sa = pl.ds(row_a, r)
        sb = pl.ds(row_b, r)
        ka = keys_ref[sa, :]
        kb = keys_ref[sb, :]
        pa = pay_ref[sa, :]
        pb = pay_ref[sb, :]
        less = _lex_less(kb, pb, ka, pa)
        swap = jnp.logical_xor(less, jnp.logical_not(up))
        nka = jnp.where(swap, kb, ka)
        nkb = jnp.where(swap, ka, kb)
        npa = jnp.where(swap, pb, pa)
        npb = jnp.where(swap, pa, pb)
        if fuse_near:
            upm = jnp.broadcast_to(up, (r, 1))
            nka, npa = _reg_finish(nka, npa, r, r // 2, upm)
            nkb, npb = _reg_finish(nkb, npb, r, r // 2, upm)
        keys_ref[sa, :] = nka
        keys_ref[sb, :] = nkb
        pay_ref[sa, :] = npa
        pay_ref[sb, :] = npb

    def body(t, carry):
        one_pair(t)
        return carry

    lax.fori_loop(0, npair, body, 0)


def _far2_pass(keys_ref, pay_ref, h, r2, j, k, flip):
    j2 = j // 2
    nbt = j2 // r2
    niter = (h // (2 * j)) * nbt

    def exchange(ka, pa, kb, pb, up):
        less = _lex_less(kb, pb, ka, pa)
        swap = jnp.logical_xor(less, jnp.logical_not(up))
        return (jnp.where(swap, kb, ka), jnp.where(swap, pb, pa),
                jnp.where(swap, ka, kb), jnp.where(swap, pa, pb))

    def body(t, carry):
        off = t & (nbt - 1)
        blk = t >> (nbt.bit_length() - 1)
        row0 = blk * (2 * j) + off * r2
        up = ((row0 & k) == 0) != flip
        sl = [pl.ds(row0, r2), pl.ds(row0 + j2, r2),
              pl.ds(row0 + j, r2), pl.ds(row0 + j + j2, r2)]
        kt = [keys_ref[s, :] for s in sl]
        pt = [pay_ref[s, :] for s in sl]
        kt[0], pt[0], kt[2], pt[2] = exchange(kt[0], pt[0], kt[2], pt[2], up)
        kt[1], pt[1], kt[3], pt[3] = exchange(kt[1], pt[1], kt[3], pt[3], up)
        kt[0], pt[0], kt[1], pt[1] = exchange(kt[0], pt[0], kt[1], pt[1], up)
        kt[2], pt[2], kt[3], pt[3] = exchange(kt[2], pt[2], kt[3], pt[3], up)
        for i, s in enumerate(sl):
            keys_ref[s, :] = kt[i]
            pay_ref[s, :] = pt[i]
        return carry

    lax.fori_loop(0, niter, body, 0)


def _merge_region(keys_ref, pay_ref, h, r, k, flip, span=None):
    j = (h if span is None else span) >> 1
    while (j >> 1) > r:
        _far2_pass(keys_ref, pay_ref, h, r // 2, j, k, flip)
        j >>= 2
    while j > r:
        _far_pass(keys_ref, pay_ref, h, r, j, k, flip, False)
        j >>= 1
    if j == r:
        _far_pass(keys_ref, pay_ref, h, r, r, k, flip, True)
    else:
        iota = lax.broadcasted_iota(jnp.int32, (h, 1), 0)

        def body(t, carry):
            kv = keys_ref[...]
            pv = pay_ref[...]
            up = ((iota & k) == 0) != flip
            kv, pv = _reg_finish(kv, pv, h, j, up)
            keys_ref[...] = kv
            pay_ref[...] = pv
            return carry

        lax.fori_loop(0, 1, body, 0)


def _phase_a_impl(h, r, x_ref, m_ref, v_ref, w_ref, b_ref, gb_ref,
                  keys_out, pay_out, kv_ref, pv_ref, sem1, sem2):
    hh = pl.program_id(0)
    base = hh * h
    c1 = pltpu.make_async_copy(x_ref.at[pl.ds(base, h)], kv_ref, sem1)
    c2 = pltpu.make_async_copy(gb_ref.at[pl.ds(base, h)], pv_ref, sem2)
    c1.start()
    c2.start()
    c1.wait()
    c2.wait()

    den = jnp.sqrt(v_ref[...] + _EPS)
    _init_pass(kv_ref, pv_ref, h, r, m_ref[...], den, w_ref[...], b_ref[...])

    logr = r.bit_length() - 1
    logh = h.bit_length() - 1
    for s in range(logr + 1, logh + 1):
        k = 1 << s
        flip = (hh == 1) if k == h else False
        _merge_region(kv_ref, pv_ref, h, r, k, flip, span=k)

    o1 = pltpu.make_async_copy(kv_ref, keys_out.at[pl.ds(base, h)], sem1)
    o2 = pltpu.make_async_copy(pv_ref, pay_out.at[pl.ds(base, h)], sem2)
    o1.start()
    o2.start()
    o1.wait()
    o2.wait()


def _phase_c_impl(h, r, cb, keys_ref, pay_ref, out_ref,
                  kv_ref, pv_ref, pk_ref, pp_ref, sem1, sem2, psems):
    hh = pl.program_id(0)
    base = hh * h
    pbase = (1 - hh) * h
    c1 = pltpu.make_async_copy(keys_ref.at[pl.ds(base, h)], kv_ref, sem1)
    c2 = pltpu.make_async_copy(pay_ref.at[pl.ds(base, h)], pv_ref, sem2)
    c1.start()
    c2.start()
    c1.wait()
    c2.wait()

    nch = h // cb
    is_hi = hh == 1

    def fetch_copies(c, buf):
        rows = pl.ds(pbase + c * cb, cb)
        k_c = pltpu.make_async_copy(keys_ref.at[rows], pk_ref.at[buf],
                                    psems.at[buf, 0])
        p_c = pltpu.make_async_copy(pay_ref.at[rows], pp_ref.at[buf],
                                    psems.at[buf, 1])
        return k_c, p_c

    def start_fetch(c, buf):
        k_c, p_c = fetch_copies(c, buf)
        k_c.start()
        p_c.start()

    def wait_fetch(c, buf):
        k_c, p_c = fetch_copies(c, buf)
        k_c.wait()
        p_c.wait()

    start_fetch(0, 0)

    def cross_body(c, carry):
        buf = lax.rem(c, 2)
        nbuf = lax.rem(c + 1, 2)

        @pl.when(c + 1 < nch)
        def _():
            start_fetch(c + 1, nbuf)

        wait_fetch(c, buf)
        nt = cb // r

        def tile_body(t, carry2):
            rows = pl.ds(c * cb + t * r, r)
            prow = pl.ds(t * r, r)
            ko = kv_ref[rows, :]
            po = pv_ref[rows, :]
            kp = pk_ref[buf, prow, :]
            pp = pp_ref[buf, prow, :]
            take = jnp.logical_xor(_lex_less(kp, pp, ko, po), is_hi)
            kv_ref[rows, :] = jnp.where(take, kp, ko)
            pv_ref[rows, :] = jnp.where(take, pp, po)
            return carry2

        lax.fori_loop(0, nt, tile_body, 0)
        return carry

    lax.fori_loop(0, nch, cross_body, 0)

    _merge_region(kv_ref, pv_ref, h, r, 2 * h, False)

    o2 = pltpu.make_async_copy(pv_ref, out_ref.at[pl.ds(base, h)], sem2)
    o2.start()
    o2.wait()


def _run2(x, bn_weight, bn_bias, running_mean, running_var, gauss_point,
          r=64, cb=2048, interpret=False):
    n, d = x.shape
    h = n // 2
    f32 = jnp.float32
    gb = jnp.broadcast_to(gauss_point[:, None], (n, d))
    m2 = running_mean.reshape(1, d)
    v2 = running_var.reshape(1, d)
    w2 = bn_weight.reshape(1, d)
    b2 = bn_bias.reshape(1, d)

    hbm = pl.BlockSpec(memory_space=pltpu.MemorySpace.HBM)
    vsmall = pl.BlockSpec((1, d), lambda hh: (0, 0))

    keys1, pay1 = pl.pallas_call(
        functools.partial(_phase_a_impl, h, r),
        grid=(2,),
        in_specs=[hbm, vsmall, vsmall, vsmall, vsmall, hbm],
        out_specs=[hbm, hbm],
        out_shape=[jax.ShapeDtypeStruct((n, d), f32),
                   jax.ShapeDtypeStruct((n, d), f32)],
        scratch_shapes=[pltpu.VMEM((h, d), f32), pltpu.VMEM((h, d), f32),
                        pltpu.SemaphoreType.DMA, pltpu.SemaphoreType.DMA],
        compiler_params=pltpu.CompilerParams(
            dimension_semantics=("arbitrary",),
        ),
        interpret=interpret,
    )(x, m2, v2, w2, b2, gb)

    out = pl.pallas_call(
        functools.partial(_phase_c_impl, h, r, cb),
        grid=(2,),
        in_specs=[hbm, hbm],
        out_specs=hbm,
        out_shape=jax.ShapeDtypeStruct((n, d), f32),
        scratch_shapes=[pltpu.VMEM((h, d), f32), pltpu.VMEM((h, d), f32),
                        pltpu.VMEM((2, cb, d), f32),
                        pltpu.VMEM((2, cb, d), f32),
                        pltpu.SemaphoreType.DMA, pltpu.SemaphoreType.DMA,
                        pltpu.SemaphoreType.DMA((2, 2))],
        compiler_params=pltpu.CompilerParams(
            dimension_semantics=("arbitrary",),
        ),
        interpret=interpret,
    )(keys1, pay1)
    return out


def kernel(x, bn_weight, bn_bias, running_mean, running_var, gauss_point):
    return _run2(x, bn_weight, bn_bias, running_mean, running_var,
                 gauss_point)
